# baseline (speedup 1.0000x reference)
import functools

import jax
import jax.numpy as jnp
from jax import lax
from jax.experimental import pallas as pl
from jax.experimental.pallas import tpu as pltpu
from jax.experimental.pallas import tpu_sc as plsc

N = 16384
D = 64
PAIRS = 50000
NUM_TABLES = 4
L = 16
CH = 128


@functools.cache
def _build(nw: int, nc: int):
    C = N // nw

    mesh = plsc.VectorSubcoreMesh(core_axis_name="c", subcore_axis_name="s")

    @functools.partial(
        pl.kernel,
        out_type=jax.ShapeDtypeStruct((N, 2 * D), jnp.float32),
        mesh=mesh,
        compiler_params=pltpu.CompilerParams(use_tc_tiling_on_sc=True,
                                             needs_layout_passes=False),
        scratch_types=[
            pltpu.VMEM((C,), jnp.int32),
            pltpu.VMEM((C,), jnp.int32),
            pltpu.VMEM((C,), jnp.int32),
            pltpu.VMEM((C,), jnp.int32),
            pltpu.VMEM((CH, 2 * D), jnp.float32),
            pltpu.VMEM((C, 2 * D), jnp.float32),
            pltpu.SemaphoreType.DMA,
        ],
    )
    def hetero_gather(x_hbm, t_hbm, w0, w1, w2, w3, out_hbm,
                      x_v, t_v, idx_v, odd_v, blk_v, acc_v, sem):
        wid = lax.axis_index("s") * nc + lax.axis_index("c")
        base = wid * C
        pltpu.sync_copy(x_hbm.at[pl.ds(base, C)], x_v)
        pltpu.sync_copy(t_hbm.at[pl.ds(base, C)], t_v)

        lanes = lax.iota(jnp.int32, L)
        tables = [w0, w1, w2, w3]
        for t in range(NUM_TABLES):
            def mkidx(g, _, t=t):
                s = pl.ds(g * L, L)
                xv = x_v[s]
                m = t_v[s] == t
                sp = (wid * 131 + g * 17 + lanes * 7) % PAIRS
                idx_v[s] = jnp.where(m, xv >> 1, sp)
                odd_v[s] = jnp.where(m, xv & 1, 0)
                return 0
            lax.fori_loop(0, C // L, mkidx, 0, unroll=4)

            for k4 in range(C // CH):
                pltpu.async_copy(
                    tables[t].at[idx_v.at[pl.ds(k4 * CH, CH)]],
                    blk_v, sem).wait()

                def extract(q, _, t=t, k4=k4):
                    s = pl.ds(k4 * CH + q * L, L)
                    oddv = odd_v[s]
                    mt = t_v[s] == t
                    rowv = q * L + lanes
                    posv = k4 * CH + q * L + lanes

                    def colb(c, _):
                        cv = jnp.full((L,), 0, jnp.int32) + c
                        v = plsc.load_gather(blk_v, [rowv, cv + D * oddv])
                        plsc.store_scatter(acc_v, [posv, cv], v, mask=mt)
                        return 0
                    lax.fori_loop(0, D, colb, 0, unroll=4)
                    return 0
                lax.fori_loop(0, CH // L, extract, 0)

        pltpu.sync_copy(acc_v, out_hbm.at[pl.ds(base, C)])

    return hetero_gather


def kernel(x, types, W0, W1, W2, W3):
    info = plsc.get_sparse_core_info()
    nw = info.num_cores * info.num_subcores
    fn = _build(nw, info.num_cores)
    tbls = [W.reshape(PAIRS, 2 * D) for W in (W0, W1, W2, W3)]
    out = fn(x.astype(jnp.int32), types.astype(jnp.int32), *tbls)
    return out[:, :D]
